# baseline (speedup 1.0000x reference)
import functools
import jax
import jax.numpy as jnp
from jax import lax
from jax.experimental import pallas as pl
from jax.experimental.pallas import tpu as pltpu
from jax.experimental.pallas import tpu_sc as plsc

N = 100000
H = 128
E = 100000

NC = 2
NS = 16
L = 16
SEG = 4096
NP = 13
NPAD = NP * NC * SEG
EC = 6272
K = 32
DUMMY_DST = 1 << 20
TRASH = EC + 3 * K
CMPSZ = EC + 3 * K + 16

_ROWS_PER_SUB = SEG // NS
_ZCH = 64


def _prep_body(x_ref, h_ref, wf_ref, uf_ref, bias_ref, p_ref, q_ref):
    p_ref[...] = (
        jnp.dot(x_ref[...], wf_ref[...], precision=lax.Precision.HIGHEST,
                preferred_element_type=jnp.float32)
        + bias_ref[...]
    )
    q_ref[...] = jnp.dot(h_ref[...], uf_ref[...],
                         precision=lax.Precision.HIGHEST,
                         preferred_element_type=jnp.float32)


def _final_body(x_ref, cred_ref, hagg_ref, wiou_ref, uiou_ref, biou_ref,
                hout_ref, cout_ref):
    iou = (
        jnp.dot(x_ref[...], wiou_ref[...], precision=lax.Precision.HIGHEST,
                preferred_element_type=jnp.float32)
        + jnp.dot(hagg_ref[...], uiou_ref[...],
                  precision=lax.Precision.HIGHEST,
                  preferred_element_type=jnp.float32)
        + biou_ref[...]
    )
    i = jax.nn.sigmoid(iou[:, :H])
    o = jax.nn.sigmoid(iou[:, H:2 * H])
    u = jnp.tanh(iou[:, 2 * H:])
    c_new = i * u + cred_ref[...]
    hout_ref[...] = o * jnp.tanh(c_new)
    cout_ref[...] = c_new


def _edge_kernel(dst_hbm, src_hbm, h_hbm, c_hbm, p_hbm, q_hbm,
                 credp_hbm, haggp_hbm,
                 dst_v, src_v, cmp_v,
                 idx_s0, idx_d0, idx_dl0, bh0, bc0, bq0, bp0,
                 idx_s1, idx_d1, idx_dl1, bh1, bc1, bq1, bp1,
                 zeros_v, scr16, acc_c, acc_h, sem0, sem1):
    cid = lax.axis_index("c")
    sid = lax.axis_index("s")
    sets = (
        (idx_s0, idx_d0, idx_dl0, bh0, bc0, bq0, bp0, sem0),
        (idx_s1, idx_d1, idx_dl1, bh1, bc1, bq1, bp1, sem1),
    )

    pltpu.sync_copy(dst_hbm.at[sid], dst_v)
    pltpu.sync_copy(src_hbm.at[sid], src_v)

    def zinit(i, carry):
        r = i // (H // L)
        col = pl.multiple_of((i % (H // L)) * L, L)
        zeros_v[r, pl.ds(col, L)] = jnp.zeros((L,), jnp.float32)
        return carry

    lax.fori_loop(0, _ZCH * (H // L), zinit, 0)

    def one_pass(p, carry):
        base = (p * NC + cid) * SEG

        for z in range(_ROWS_PER_SUB // _ZCH):
            row0 = sid * _ROWS_PER_SUB + z * _ZCH
            pltpu.sync_copy(zeros_v, acc_c.at[pl.ds(row0, _ZCH)])
            pltpu.sync_copy(zeros_v, acc_h.at[pl.ds(row0, _ZCH)])
        plsc.subcore_barrier()

        def filt(j, cnt):
            off = pl.multiple_of(j * L, L)
            iot = lax.iota(jnp.int32, L)
            ids = iot + off
            d = dst_v[pl.ds(off, L)]
            m = jnp.logical_and(d >= base, d < base + SEG)
            px = jnp.where(m, jnp.int32(1), jnp.int32(0))
            for k in (1, 2, 4, 8):
                scr16[...] = px
                g = plsc.load_gather(scr16, [jnp.maximum(iot - k, 0)])
                px = px + jnp.where(iot >= k, g, 0)
            pos = jnp.where(m, cnt + px - 1, jnp.int32(TRASH))
            plsc.store_scatter(cmp_v, [pos], ids)
            return cnt + px[15]

        m_cnt = lax.fori_loop(0, EC // L, filt, jnp.int32(0))

        for t in range(2 * K // L):
            pos = m_cnt + t * L + lax.iota(jnp.int32, L)
            plsc.store_scatter(cmp_v, [pos], jnp.full((L,), EC - 1, jnp.int32))

        n_chunks = (m_cnt + (K - 1)) // K
        n_pairs = jnp.maximum((n_chunks + 1) // 2, 1)

        def stage_fire(ck, b):
            idx_s, idx_d, idx_dl, bh, bc, bq, bp, sem = sets[b]
            for t in range(K // L):
                ids = cmp_v[pl.ds(ck * K + t * L, L)]
                ids = jnp.clip(ids, 0, EC - 1)
                sidx = plsc.load_gather(src_v, [ids])
                d = plsc.load_gather(dst_v, [ids])
                idx_s[pl.ds(t * L, L)] = sidx
                idx_d[pl.ds(t * L, L)] = jnp.minimum(d, N - 1)
                idx_dl[pl.ds(t * L, L)] = jnp.clip(d - base, 0, SEG)
            pltpu.async_copy(h_hbm.at[idx_s], bh, sem)
            pltpu.async_copy(c_hbm.at[idx_s], bc, sem)
            pltpu.async_copy(q_hbm.at[idx_s], bq, sem)
            pltpu.async_copy(p_hbm.at[idx_d], bp, sem)

        def wait_set(b):
            idx_s, idx_d, idx_dl, bh, bc, bq, bp, sem = sets[b]
            pltpu.make_async_copy(h_hbm.at[idx_s], bh, sem).wait()
            pltpu.make_async_copy(c_hbm.at[idx_s], bc, sem).wait()
            pltpu.make_async_copy(q_hbm.at[idx_s], bq, sem).wait()
            pltpu.make_async_copy(p_hbm.at[idx_d], bp, sem).wait()

        def compute_scatter(b):
            idx_s, idx_d, idx_dl, bh, bc, bq, bp, sem = sets[b]

            def fuse(r, carry3):
                for t in range(H // L):
                    col = t * L
                    pv = bp[r, pl.ds(col, L)]
                    qv = bq[r, pl.ds(col, L)]
                    cv = bc[r, pl.ds(col, L)]
                    f = 1.0 / (1.0 + jnp.exp(-(pv + qv)))
                    bc[r, pl.ds(col, L)] = f * cv
                return carry3

            lax.fori_loop(0, K, fuse, 0)
            pltpu.sync_copy(bc, acc_c.at[idx_dl], add=True)
            pltpu.sync_copy(bh, acc_h.at[idx_dl], add=True)

        stage_fire(0, 0)

        def pair(i2, carry2):
            c0 = pl.multiple_of(i2 * 2, 2)
            wait_set(0)
            stage_fire(c0 + 1, 1)
            compute_scatter(0)
            wait_set(1)
            stage_fire(c0 + 2, 0)
            compute_scatter(1)
            return carry2

        lax.fori_loop(0, n_pairs, pair, 0)
        wait_set(0)
        plsc.subcore_barrier()

        row0 = sid * _ROWS_PER_SUB
        pltpu.sync_copy(acc_c.at[pl.ds(row0, _ROWS_PER_SUB)],
                        credp_hbm.at[pl.ds(base + row0, _ROWS_PER_SUB)])
        pltpu.sync_copy(acc_h.at[pl.ds(row0, _ROWS_PER_SUB)],
                        haggp_hbm.at[pl.ds(base + row0, _ROWS_PER_SUB)])
        plsc.subcore_barrier()
        return carry

    lax.fori_loop(0, NP, one_pass, 0)


@jax.jit
def kernel(x, h, c, W_iou, U_iou, b_iou, W_f_w, W_f_b, b_f, U_f_w, U_f_b,
           edge_index):
    R = 1000
    grid = N // R

    bias_f = (W_f_b + b_f[0] + U_f_b).reshape(1, H)
    p_arr, q_arr = pl.pallas_call(
        _prep_body,
        grid=(grid,),
        in_specs=[
            pl.BlockSpec((R, H), lambda i: (i, 0)),
            pl.BlockSpec((R, H), lambda i: (i, 0)),
            pl.BlockSpec((H, H), lambda i: (0, 0)),
            pl.BlockSpec((H, H), lambda i: (0, 0)),
            pl.BlockSpec((1, H), lambda i: (0, 0)),
        ],
        out_specs=[
            pl.BlockSpec((R, H), lambda i: (i, 0)),
            pl.BlockSpec((R, H), lambda i: (i, 0)),
        ],
        out_shape=[
            jax.ShapeDtypeStruct((N, H), jnp.float32),
            jax.ShapeDtypeStruct((N, H), jnp.float32),
        ],
    )(x, h, W_f_w.T, U_f_w.T, bias_f)

    src = edge_index[0].reshape(NS, E // NS)
    dst = edge_index[1].reshape(NS, E // NS)
    pad = ((0, 0), (0, EC - E // NS))
    src_p = jnp.pad(src, pad, constant_values=0)
    dst_p = jnp.pad(dst, pad, constant_values=DUMMY_DST)

    mesh = plsc.VectorSubcoreMesh(core_axis_name="c", subcore_axis_name="s")
    credp, haggp = pl.kernel(
        _edge_kernel,
        out_type=[
            jax.ShapeDtypeStruct((NPAD, H), jnp.float32),
            jax.ShapeDtypeStruct((NPAD, H), jnp.float32),
        ],
        mesh=mesh,
        compiler_params=pltpu.CompilerParams(needs_layout_passes=False),
        scratch_types=(
            [
                pltpu.VMEM((EC,), jnp.int32),
                pltpu.VMEM((EC,), jnp.int32),
                pltpu.VMEM((CMPSZ,), jnp.int32),
            ]
            + [
                pltpu.VMEM((K,), jnp.int32),
                pltpu.VMEM((K,), jnp.int32),
                pltpu.VMEM((K,), jnp.int32),
                pltpu.VMEM((K, H), jnp.float32),
                pltpu.VMEM((K, H), jnp.float32),
                pltpu.VMEM((K, H), jnp.float32),
                pltpu.VMEM((K, H), jnp.float32),
            ] * 2
            + [
                pltpu.VMEM((_ZCH, H), jnp.float32),
                pltpu.VMEM((L,), jnp.int32),
                pltpu.VMEM_SHARED((SEG + 8, H), jnp.float32),
                pltpu.VMEM_SHARED((SEG + 8, H), jnp.float32),
                pltpu.SemaphoreType.DMA,
                pltpu.SemaphoreType.DMA,
            ]
        ),
    )(dst_p, src_p, h, c, p_arr, q_arr)

    c_red = credp[:N]
    h_agg = haggp[:N]

    h_new, c_new = pl.pallas_call(
        _final_body,
        grid=(grid,),
        in_specs=[
            pl.BlockSpec((R, H), lambda i: (i, 0)),
            pl.BlockSpec((R, H), lambda i: (i, 0)),
            pl.BlockSpec((R, H), lambda i: (i, 0)),
            pl.BlockSpec((H, 3 * H), lambda i: (0, 0)),
            pl.BlockSpec((H, 3 * H), lambda i: (0, 0)),
            pl.BlockSpec((1, 3 * H), lambda i: (0, 0)),
        ],
        out_specs=[
            pl.BlockSpec((R, H), lambda i: (i, 0)),
            pl.BlockSpec((R, H), lambda i: (i, 0)),
        ],
        out_shape=[
            jax.ShapeDtypeStruct((N, H), jnp.float32),
            jax.ShapeDtypeStruct((N, H), jnp.float32),
        ],
    )(x, c_red, h_agg, W_iou.T, U_iou.T, b_iou)

    return h_new, c_new


# speedup vs baseline: 1.0041x; 1.0041x over previous
import functools
import jax
import jax.numpy as jnp
from jax import lax
from jax.experimental import pallas as pl
from jax.experimental.pallas import tpu as pltpu
from jax.experimental.pallas import tpu_sc as plsc

N = 100000
H = 128
E = 100000

NC = 2
NS = 16
L = 16
SEG = 4096
NP = 13
NPAD = NP * NC * SEG
EC = 6272
K = 32
DUMMY_DST = 1 << 20
TRASH = EC + 3 * K
CMPSZ = EC + 3 * K + 16

_ROWS_PER_SUB = SEG // NS
_ZCH = 64


def _prep_body(x_ref, h_ref, wf_ref, uf_ref, bias_ref, p_ref, q_ref):
    p_ref[...] = (
        jnp.dot(x_ref[...], wf_ref[...], precision=lax.Precision.HIGHEST,
                preferred_element_type=jnp.float32)
        + bias_ref[...]
    )
    q_ref[...] = jnp.dot(h_ref[...], uf_ref[...],
                         precision=lax.Precision.HIGHEST,
                         preferred_element_type=jnp.float32)


def _final_body(x_ref, cred_ref, hagg_ref, wiou_ref, uiou_ref, biou_ref,
                hout_ref, cout_ref):
    iou = (
        jnp.dot(x_ref[...], wiou_ref[...], precision=lax.Precision.HIGHEST,
                preferred_element_type=jnp.float32)
        + jnp.dot(hagg_ref[...], uiou_ref[...],
                  precision=lax.Precision.HIGHEST,
                  preferred_element_type=jnp.float32)
        + biou_ref[...]
    )
    i = jax.nn.sigmoid(iou[:, :H])
    o = jax.nn.sigmoid(iou[:, H:2 * H])
    u = jnp.tanh(iou[:, 2 * H:])
    c_new = i * u + cred_ref[...]
    hout_ref[...] = o * jnp.tanh(c_new)
    cout_ref[...] = c_new


def _edge_kernel(dst_hbm, src_hbm, h_hbm, c_hbm, p_hbm, q_hbm,
                 credp_hbm, haggp_hbm,
                 dst_v, src_v, cmp_v,
                 idx_s0, idx_d0, idx_dl0, bh0, bc0, bq0, bp0,
                 idx_s1, idx_d1, idx_dl1, bh1, bc1, bq1, bp1,
                 zeros_v, scr16, acc_c, acc_h, sem0, sem1):
    cid = lax.axis_index("c")
    sid = lax.axis_index("s")
    sets = (
        (idx_s0, idx_d0, idx_dl0, bh0, bc0, bq0, bp0, sem0),
        (idx_s1, idx_d1, idx_dl1, bh1, bc1, bq1, bp1, sem1),
    )

    pltpu.sync_copy(dst_hbm.at[sid], dst_v)
    pltpu.sync_copy(src_hbm.at[sid], src_v)

    def zinit(i, carry):
        r = i // (H // L)
        col = pl.multiple_of((i % (H // L)) * L, L)
        zeros_v[r, pl.ds(col, L)] = jnp.zeros((L,), jnp.float32)
        return carry

    lax.fori_loop(0, _ZCH * (H // L), zinit, 0)

    def one_pass(p, carry):
        base = (p * NC + cid) * SEG

        for z in range(_ROWS_PER_SUB // _ZCH):
            row0 = sid * _ROWS_PER_SUB + z * _ZCH
            pltpu.sync_copy(zeros_v, acc_c.at[pl.ds(row0, _ZCH)])
            pltpu.sync_copy(zeros_v, acc_h.at[pl.ds(row0, _ZCH)])
        plsc.subcore_barrier()

        def filt(j, cnt):
            off = pl.multiple_of(j * L, L)
            iot = lax.iota(jnp.int32, L)
            ids = iot + off
            d = dst_v[pl.ds(off, L)]
            m = jnp.logical_and(d >= base, d < base + SEG)
            px = jnp.where(m, jnp.int32(1), jnp.int32(0))
            for k in (1, 2, 4, 8):
                scr16[...] = px
                g = plsc.load_gather(scr16, [jnp.maximum(iot - k, 0)])
                px = px + jnp.where(iot >= k, g, 0)
            pos = jnp.where(m, cnt + px - 1, jnp.int32(TRASH))
            plsc.store_scatter(cmp_v, [pos], ids)
            return cnt + px[15]

        m_cnt = lax.fori_loop(0, EC // L, filt, jnp.int32(0))

        for t in range(2 * K // L):
            pos = m_cnt + t * L + lax.iota(jnp.int32, L)
            plsc.store_scatter(cmp_v, [pos], jnp.full((L,), EC - 1, jnp.int32))

        n_chunks = (m_cnt + (K - 1)) // K
        n_pairs = jnp.maximum((n_chunks + 1) // 2, 1)

        def stage_fire(ck, b):
            idx_s, idx_d, idx_dl, bh, bc, bq, bp, sem = sets[b]
            off = pl.multiple_of(ck * K, K)
            for t in range(K // L):
                ids = cmp_v[pl.ds(off + t * L, L)]
                ids = jnp.clip(ids, 0, EC - 1)
                sidx = plsc.load_gather(src_v, [ids])
                d = plsc.load_gather(dst_v, [ids])
                idx_s[pl.ds(t * L, L)] = sidx
                idx_d[pl.ds(t * L, L)] = jnp.minimum(d, N - 1)
                idx_dl[pl.ds(t * L, L)] = jnp.clip(d - base, 0, SEG)
            pltpu.async_copy(h_hbm.at[idx_s], bh, sem)
            pltpu.async_copy(c_hbm.at[idx_s], bc, sem)
            pltpu.async_copy(q_hbm.at[idx_s], bq, sem)
            pltpu.async_copy(p_hbm.at[idx_d], bp, sem)

        def wait_set(b):
            idx_s, idx_d, idx_dl, bh, bc, bq, bp, sem = sets[b]
            pltpu.make_async_copy(h_hbm.at[idx_s], bh, sem).wait()
            pltpu.make_async_copy(c_hbm.at[idx_s], bc, sem).wait()
            pltpu.make_async_copy(q_hbm.at[idx_s], bq, sem).wait()
            pltpu.make_async_copy(p_hbm.at[idx_d], bp, sem).wait()

        def compute_scatter(b):
            idx_s, idx_d, idx_dl, bh, bc, bq, bp, sem = sets[b]

            def fuse(r, carry3):
                for t in range(H // L):
                    col = t * L
                    pv = bp[r, pl.ds(col, L)]
                    qv = bq[r, pl.ds(col, L)]
                    cv = bc[r, pl.ds(col, L)]
                    f = 1.0 / (1.0 + jnp.exp(-(pv + qv)))
                    bc[r, pl.ds(col, L)] = f * cv
                return carry3

            lax.fori_loop(0, K, fuse, 0)
            pltpu.sync_copy(bc, acc_c.at[idx_dl], add=True)
            pltpu.sync_copy(bh, acc_h.at[idx_dl], add=True)

        stage_fire(0, 0)

        def pair(i2, carry2):
            c0 = pl.multiple_of(i2 * 2, 2)
            wait_set(0)
            stage_fire(c0 + 1, 1)
            compute_scatter(0)
            wait_set(1)
            stage_fire(c0 + 2, 0)
            compute_scatter(1)
            return carry2

        lax.fori_loop(0, n_pairs, pair, 0)
        wait_set(0)
        plsc.subcore_barrier()

        row0 = sid * _ROWS_PER_SUB
        pltpu.sync_copy(acc_c.at[pl.ds(row0, _ROWS_PER_SUB)],
                        credp_hbm.at[pl.ds(base + row0, _ROWS_PER_SUB)])
        pltpu.sync_copy(acc_h.at[pl.ds(row0, _ROWS_PER_SUB)],
                        haggp_hbm.at[pl.ds(base + row0, _ROWS_PER_SUB)])
        plsc.subcore_barrier()
        return carry

    lax.fori_loop(0, NP, one_pass, 0)


@jax.jit
def kernel(x, h, c, W_iou, U_iou, b_iou, W_f_w, W_f_b, b_f, U_f_w, U_f_b,
           edge_index):
    R = 1000
    grid = N // R

    bias_f = (W_f_b + b_f[0] + U_f_b).reshape(1, H)
    p_arr, q_arr = pl.pallas_call(
        _prep_body,
        grid=(grid,),
        in_specs=[
            pl.BlockSpec((R, H), lambda i: (i, 0)),
            pl.BlockSpec((R, H), lambda i: (i, 0)),
            pl.BlockSpec((H, H), lambda i: (0, 0)),
            pl.BlockSpec((H, H), lambda i: (0, 0)),
            pl.BlockSpec((1, H), lambda i: (0, 0)),
        ],
        out_specs=[
            pl.BlockSpec((R, H), lambda i: (i, 0)),
            pl.BlockSpec((R, H), lambda i: (i, 0)),
        ],
        out_shape=[
            jax.ShapeDtypeStruct((N, H), jnp.float32),
            jax.ShapeDtypeStruct((N, H), jnp.float32),
        ],
    )(x, h, W_f_w.T, U_f_w.T, bias_f)

    src = edge_index[0].reshape(NS, E // NS)
    dst = edge_index[1].reshape(NS, E // NS)
    pad = ((0, 0), (0, EC - E // NS))
    src_p = jnp.pad(src, pad, constant_values=0)
    dst_p = jnp.pad(dst, pad, constant_values=DUMMY_DST)

    mesh = plsc.VectorSubcoreMesh(core_axis_name="c", subcore_axis_name="s")
    credp, haggp = pl.kernel(
        _edge_kernel,
        out_type=[
            jax.ShapeDtypeStruct((NPAD, H), jnp.float32),
            jax.ShapeDtypeStruct((NPAD, H), jnp.float32),
        ],
        mesh=mesh,
        compiler_params=pltpu.CompilerParams(needs_layout_passes=False),
        scratch_types=(
            [
                pltpu.VMEM((EC,), jnp.int32),
                pltpu.VMEM((EC,), jnp.int32),
                pltpu.VMEM((CMPSZ,), jnp.int32),
            ]
            + [
                pltpu.VMEM((K,), jnp.int32),
                pltpu.VMEM((K,), jnp.int32),
                pltpu.VMEM((K,), jnp.int32),
                pltpu.VMEM((K, H), jnp.float32),
                pltpu.VMEM((K, H), jnp.float32),
                pltpu.VMEM((K, H), jnp.float32),
                pltpu.VMEM((K, H), jnp.float32),
            ] * 2
            + [
                pltpu.VMEM((_ZCH, H), jnp.float32),
                pltpu.VMEM((L,), jnp.int32),
                pltpu.VMEM_SHARED((SEG + 8, H), jnp.float32),
                pltpu.VMEM_SHARED((SEG + 8, H), jnp.float32),
                pltpu.SemaphoreType.DMA,
                pltpu.SemaphoreType.DMA,
            ]
        ),
    )(dst_p, src_p, h, c, p_arr, q_arr)

    c_red = credp[:N]
    h_agg = haggp[:N]

    h_new, c_new = pl.pallas_call(
        _final_body,
        grid=(grid,),
        in_specs=[
            pl.BlockSpec((R, H), lambda i: (i, 0)),
            pl.BlockSpec((R, H), lambda i: (i, 0)),
            pl.BlockSpec((R, H), lambda i: (i, 0)),
            pl.BlockSpec((H, 3 * H), lambda i: (0, 0)),
            pl.BlockSpec((H, 3 * H), lambda i: (0, 0)),
            pl.BlockSpec((1, 3 * H), lambda i: (0, 0)),
        ],
        out_specs=[
            pl.BlockSpec((R, H), lambda i: (i, 0)),
            pl.BlockSpec((R, H), lambda i: (i, 0)),
        ],
        out_shape=[
            jax.ShapeDtypeStruct((N, H), jnp.float32),
            jax.ShapeDtypeStruct((N, H), jnp.float32),
        ],
    )(x, c_red, h_agg, W_iou.T, U_iou.T, b_iou)

    return h_new, c_new


# speedup vs baseline: 1.5417x; 1.5354x over previous
import functools
import jax
import jax.numpy as jnp
from jax import lax
from jax.experimental import pallas as pl
from jax.experimental.pallas import tpu as pltpu
from jax.experimental.pallas import tpu_sc as plsc

N = 100000
H = 128
E = 100000

NC = 2
NS = 16
L = 16
SEG = 4608
NP = 11
NPAD = NP * NC * SEG
EC = 6272
K = 32
DUMMY_DST = 1 << 20
TRASH = EC + 3 * K
CMPSZ = EC + 3 * K + 16

_ROWS_PER_SUB = SEG // NS
_ZCH = 48


def _prep_body(x_ref, h_ref, wf_ref, uf_ref, bias_ref, p_ref, q_ref):
    p_ref[...] = (
        jnp.dot(x_ref[...], wf_ref[...], precision=lax.Precision.HIGHEST,
                preferred_element_type=jnp.float32)
        + bias_ref[...]
    )
    q_ref[...] = jnp.dot(h_ref[...], uf_ref[...],
                         precision=lax.Precision.HIGHEST,
                         preferred_element_type=jnp.float32)


def _final_body(x_ref, cred_ref, hagg_ref, wiou_ref, uiou_ref, biou_ref,
                hout_ref, cout_ref):
    iou = (
        jnp.dot(x_ref[...], wiou_ref[...], precision=lax.Precision.HIGHEST,
                preferred_element_type=jnp.float32)
        + jnp.dot(hagg_ref[...], uiou_ref[...],
                  precision=lax.Precision.HIGHEST,
                  preferred_element_type=jnp.float32)
        + biou_ref[...]
    )
    i = jax.nn.sigmoid(iou[:, :H])
    o = jax.nn.sigmoid(iou[:, H:2 * H])
    u = jnp.tanh(iou[:, 2 * H:])
    c_new = i * u + cred_ref[...]
    hout_ref[...] = o * jnp.tanh(c_new)
    cout_ref[...] = c_new


def _edge_kernel(dst_hbm, src_hbm, h_hbm, c_hbm, p_hbm, q_hbm,
                 credp_hbm, haggp_hbm,
                 dst_v, src_v, cmp_v,
                 idx_s0, idx_d0, idx_dl0, bh0, bc0, bq0, bp0,
                 idx_s1, idx_d1, idx_dl1, bh1, bc1, bq1, bp1,
                 zeros_v, scr16, acc_c, acc_h, sem0, sem1):
    cid = lax.axis_index("c")
    sid = lax.axis_index("s")
    sets = (
        (idx_s0, idx_d0, idx_dl0, bh0, bc0, bq0, bp0, sem0),
        (idx_s1, idx_d1, idx_dl1, bh1, bc1, bq1, bp1, sem1),
    )

    pltpu.sync_copy(dst_hbm.at[sid], dst_v)
    pltpu.sync_copy(src_hbm.at[sid], src_v)

    def zinit(i, carry):
        r = i // (H // L)
        col = pl.multiple_of((i % (H // L)) * L, L)
        zeros_v[r, pl.ds(col, L)] = jnp.zeros((L,), jnp.float32)
        return carry

    lax.fori_loop(0, _ZCH * (H // L), zinit, 0)

    def one_pass(p, carry):
        base = (p * NC + cid) * SEG

        for z in range(_ROWS_PER_SUB // _ZCH):
            row0 = sid * _ROWS_PER_SUB + z * _ZCH
            pltpu.sync_copy(zeros_v, acc_c.at[pl.ds(row0, _ZCH)])
            pltpu.sync_copy(zeros_v, acc_h.at[pl.ds(row0, _ZCH)])
        plsc.subcore_barrier()

        def filt(j, cnt):
            off = pl.multiple_of(j * L, L)
            iot = lax.iota(jnp.int32, L)
            ids = iot + off
            d = dst_v[pl.ds(off, L)]
            m = jnp.logical_and(d >= base, d < base + SEG)
            px = jnp.where(m, jnp.int32(1), jnp.int32(0))
            for k in (1, 2, 4, 8):
                scr16[...] = px
                g = plsc.load_gather(scr16, [jnp.maximum(iot - k, 0)])
                px = px + jnp.where(iot >= k, g, 0)
            pos = jnp.where(m, cnt + px - 1, jnp.int32(TRASH))
            plsc.store_scatter(cmp_v, [pos], ids)
            return cnt + px[15]

        m_cnt = lax.fori_loop(0, EC // L, filt, jnp.int32(0))

        for t in range(K // L):
            pos = m_cnt + t * L + lax.iota(jnp.int32, L)
            plsc.store_scatter(cmp_v, [pos], jnp.full((L,), EC - 1, jnp.int32))

        n_chunks = (m_cnt + (K - 1)) // K

        def chunk(ck, carry2):
            off = pl.multiple_of(ck * K, K)
            idx_s, idx_d, idx_dl, bh, bc, bq, bp, sem = sets[0]
            for t in range(K // L):
                ids = cmp_v[pl.ds(off + t * L, L)]
                sidx = plsc.load_gather(src_v, [ids])
                d = plsc.load_gather(dst_v, [ids])
                idx_s[pl.ds(t * L, L)] = sidx
                idx_d[pl.ds(t * L, L)] = jnp.minimum(d, N - 1)
                idx_dl[pl.ds(t * L, L)] = jnp.clip(d - base, 0, SEG)
            cp1 = pltpu.async_copy(h_hbm.at[idx_s], bh, sem)
            cp2 = pltpu.async_copy(c_hbm.at[idx_s], bc, sem)
            cp3 = pltpu.async_copy(q_hbm.at[idx_s], bq, sem)
            cp4 = pltpu.async_copy(p_hbm.at[idx_d], bp, sem)
            cp1.wait()
            cp2.wait()
            cp3.wait()
            cp4.wait()

            def fuse(r, carry3):
                for t in range(H // L):
                    col = t * L
                    pv = bp[r, pl.ds(col, L)]
                    qv = bq[r, pl.ds(col, L)]
                    cv = bc[r, pl.ds(col, L)]
                    f = 1.0 / (1.0 + jnp.exp(-(pv + qv)))
                    bc[r, pl.ds(col, L)] = f * cv
                return carry3

            lax.fori_loop(0, K, fuse, 0)
            pltpu.sync_copy(bc, acc_c.at[idx_dl], add=True)
            pltpu.sync_copy(bh, acc_h.at[idx_dl], add=True)
            return carry2

        lax.fori_loop(0, n_chunks, chunk, 0)
        plsc.subcore_barrier()

        row0 = sid * _ROWS_PER_SUB
        pltpu.sync_copy(acc_c.at[pl.ds(row0, _ROWS_PER_SUB)],
                        credp_hbm.at[pl.ds(base + row0, _ROWS_PER_SUB)])
        pltpu.sync_copy(acc_h.at[pl.ds(row0, _ROWS_PER_SUB)],
                        haggp_hbm.at[pl.ds(base + row0, _ROWS_PER_SUB)])
        plsc.subcore_barrier()
        return carry

    lax.fori_loop(0, NP, one_pass, 0)


@jax.jit
def kernel(x, h, c, W_iou, U_iou, b_iou, W_f_w, W_f_b, b_f, U_f_w, U_f_b,
           edge_index):
    R = 1000
    grid = N // R

    bias_f = (W_f_b + b_f[0] + U_f_b).reshape(1, H)
    p_arr, q_arr = pl.pallas_call(
        _prep_body,
        grid=(grid,),
        in_specs=[
            pl.BlockSpec((R, H), lambda i: (i, 0)),
            pl.BlockSpec((R, H), lambda i: (i, 0)),
            pl.BlockSpec((H, H), lambda i: (0, 0)),
            pl.BlockSpec((H, H), lambda i: (0, 0)),
            pl.BlockSpec((1, H), lambda i: (0, 0)),
        ],
        out_specs=[
            pl.BlockSpec((R, H), lambda i: (i, 0)),
            pl.BlockSpec((R, H), lambda i: (i, 0)),
        ],
        out_shape=[
            jax.ShapeDtypeStruct((N, H), jnp.float32),
            jax.ShapeDtypeStruct((N, H), jnp.float32),
        ],
    )(x, h, W_f_w.T, U_f_w.T, bias_f)

    src = edge_index[0].reshape(NS, E // NS)
    dst = edge_index[1].reshape(NS, E // NS)
    pad = ((0, 0), (0, EC - E // NS))
    src_p = jnp.pad(src, pad, constant_values=0)
    dst_p = jnp.pad(dst, pad, constant_values=DUMMY_DST)

    mesh = plsc.VectorSubcoreMesh(core_axis_name="c", subcore_axis_name="s")
    credp, haggp = pl.kernel(
        _edge_kernel,
        out_type=[
            jax.ShapeDtypeStruct((NPAD, H), jnp.float32),
            jax.ShapeDtypeStruct((NPAD, H), jnp.float32),
        ],
        mesh=mesh,
        compiler_params=pltpu.CompilerParams(needs_layout_passes=False),
        scratch_types=(
            [
                pltpu.VMEM((EC,), jnp.int32),
                pltpu.VMEM((EC,), jnp.int32),
                pltpu.VMEM((CMPSZ,), jnp.int32),
            ]
            + [
                pltpu.VMEM((K,), jnp.int32),
                pltpu.VMEM((K,), jnp.int32),
                pltpu.VMEM((K,), jnp.int32),
                pltpu.VMEM((K, H), jnp.float32),
                pltpu.VMEM((K, H), jnp.float32),
                pltpu.VMEM((K, H), jnp.float32),
                pltpu.VMEM((K, H), jnp.float32),
            ] * 2
            + [
                pltpu.VMEM((_ZCH, H), jnp.float32),
                pltpu.VMEM((L,), jnp.int32),
                pltpu.VMEM_SHARED((SEG + 8, H), jnp.float32),
                pltpu.VMEM_SHARED((SEG + 8, H), jnp.float32),
                pltpu.SemaphoreType.DMA,
                pltpu.SemaphoreType.DMA,
            ]
        ),
    )(dst_p, src_p, h, c, p_arr, q_arr)

    c_red = credp[:N]
    h_agg = haggp[:N]

    h_new, c_new = pl.pallas_call(
        _final_body,
        grid=(grid,),
        in_specs=[
            pl.BlockSpec((R, H), lambda i: (i, 0)),
            pl.BlockSpec((R, H), lambda i: (i, 0)),
            pl.BlockSpec((R, H), lambda i: (i, 0)),
            pl.BlockSpec((H, 3 * H), lambda i: (0, 0)),
            pl.BlockSpec((H, 3 * H), lambda i: (0, 0)),
            pl.BlockSpec((1, 3 * H), lambda i: (0, 0)),
        ],
        out_specs=[
            pl.BlockSpec((R, H), lambda i: (i, 0)),
            pl.BlockSpec((R, H), lambda i: (i, 0)),
        ],
        out_shape=[
            jax.ShapeDtypeStruct((N, H), jnp.float32),
            jax.ShapeDtypeStruct((N, H), jnp.float32),
        ],
    )(x, c_red, h_agg, W_iou.T, U_iou.T, b_iou)

    return h_new, c_new


# speedup vs baseline: 1.5797x; 1.0247x over previous
import functools
import jax
import jax.numpy as jnp
from jax import lax
from jax.experimental import pallas as pl
from jax.experimental.pallas import tpu as pltpu
from jax.experimental.pallas import tpu_sc as plsc

N = 100000
H = 128
E = 100000

NC = 2
NS = 16
L = 16
SEG = 5120
NP = 10
NPAD = NP * NC * SEG
EC = 6272
K = 32
DUMMY_DST = 1 << 20
TRASH = EC + K
CMPSZ = EC + K + 16

_ROWS_PER_SUB = SEG // NS
_ZCH = 32


def _prep_body(x_ref, h_ref, wf_ref, uf_ref, bias_ref, p_ref, q_ref):
    p_ref[...] = (
        jnp.dot(x_ref[...], wf_ref[...], precision=lax.Precision.HIGHEST,
                preferred_element_type=jnp.float32)
        + bias_ref[...]
    )
    q_ref[...] = jnp.dot(h_ref[...], uf_ref[...],
                         precision=lax.Precision.HIGHEST,
                         preferred_element_type=jnp.float32)


def _final_body(x_ref, cred_ref, hagg_ref, wiou_ref, uiou_ref, biou_ref,
                hout_ref, cout_ref):
    iou = (
        jnp.dot(x_ref[...], wiou_ref[...], precision=lax.Precision.HIGHEST,
                preferred_element_type=jnp.float32)
        + jnp.dot(hagg_ref[...], uiou_ref[...],
                  precision=lax.Precision.HIGHEST,
                  preferred_element_type=jnp.float32)
        + biou_ref[...]
    )
    i = jax.nn.sigmoid(iou[:, :H])
    o = jax.nn.sigmoid(iou[:, H:2 * H])
    u = jnp.tanh(iou[:, 2 * H:])
    c_new = i * u + cred_ref[...]
    hout_ref[...] = o * jnp.tanh(c_new)
    cout_ref[...] = c_new


def _edge_kernel(dst_hbm, src_hbm, h_hbm, c_hbm, p_hbm, q_hbm,
                 credp_hbm, haggp_hbm,
                 dst_v, src_v, cmp_v,
                 idx_s, idx_d, idx_dl, bh, bc, bq, bp,
                 zeros_v, scr16, acc_c, acc_h, sem):
    cid = lax.axis_index("c")
    sid = lax.axis_index("s")

    pltpu.sync_copy(dst_hbm.at[sid], dst_v)
    pltpu.sync_copy(src_hbm.at[sid], src_v)

    def zinit(i, carry):
        r = i // (H // L)
        col = pl.multiple_of((i % (H // L)) * L, L)
        zeros_v[r, pl.ds(col, L)] = jnp.zeros((L,), jnp.float32)
        return carry

    lax.fori_loop(0, _ZCH * (H // L), zinit, 0)

    def one_pass(p, carry):
        base = (p * NC + cid) * SEG

        for z in range(_ROWS_PER_SUB // _ZCH):
            row0 = sid * _ROWS_PER_SUB + z * _ZCH
            pltpu.sync_copy(zeros_v, acc_c.at[pl.ds(row0, _ZCH)])
            pltpu.sync_copy(zeros_v, acc_h.at[pl.ds(row0, _ZCH)])
        plsc.subcore_barrier()

        def filt(j, cnt):
            off = pl.multiple_of(j * L, L)
            iot = lax.iota(jnp.int32, L)
            ids = iot + off
            d = dst_v[pl.ds(off, L)]
            m = jnp.logical_and(d >= base, d < base + SEG)
            px = jnp.where(m, jnp.int32(1), jnp.int32(0))
            for k in (1, 2, 4, 8):
                scr16[...] = px
                g = plsc.load_gather(scr16, [jnp.maximum(iot - k, 0)])
                px = px + jnp.where(iot >= k, g, 0)
            pos = jnp.where(m, cnt + px - 1, jnp.int32(TRASH))
            plsc.store_scatter(cmp_v, [pos], ids)
            return cnt + px[15]

        m_cnt = lax.fori_loop(0, EC // L, filt, jnp.int32(0))

        for t in range(K // L):
            pos = m_cnt + t * L + lax.iota(jnp.int32, L)
            plsc.store_scatter(cmp_v, [pos], jnp.full((L,), EC - 1, jnp.int32))

        n_chunks = (m_cnt + (K - 1)) // K

        def chunk(ck, carry2):
            off = pl.multiple_of(ck * K, K)
            for t in range(K // L):
                ids = cmp_v[pl.ds(off + t * L, L)]
                sidx = plsc.load_gather(src_v, [ids])
                d = plsc.load_gather(dst_v, [ids])
                idx_s[pl.ds(t * L, L)] = sidx
                idx_d[pl.ds(t * L, L)] = jnp.minimum(d, N - 1)
                idx_dl[pl.ds(t * L, L)] = jnp.clip(d - base, 0, SEG)
            cp1 = pltpu.async_copy(h_hbm.at[idx_s], bh, sem)
            cp2 = pltpu.async_copy(c_hbm.at[idx_s], bc, sem)
            cp3 = pltpu.async_copy(q_hbm.at[idx_s], bq, sem)
            cp4 = pltpu.async_copy(p_hbm.at[idx_d], bp, sem)
            cp1.wait()
            cp2.wait()
            cp3.wait()
            cp4.wait()

            def fuse(r, carry3):
                for t in range(H // L):
                    col = t * L
                    pv = bp[r, pl.ds(col, L)]
                    qv = bq[r, pl.ds(col, L)]
                    cv = bc[r, pl.ds(col, L)]
                    f = 1.0 / (1.0 + jnp.exp(-(pv + qv)))
                    bc[r, pl.ds(col, L)] = f * cv
                return carry3

            lax.fori_loop(0, K, fuse, 0)
            pltpu.sync_copy(bc, acc_c.at[idx_dl], add=True)
            pltpu.sync_copy(bh, acc_h.at[idx_dl], add=True)
            return carry2

        lax.fori_loop(0, n_chunks, chunk, 0)
        plsc.subcore_barrier()

        row0 = sid * _ROWS_PER_SUB
        pltpu.sync_copy(acc_c.at[pl.ds(row0, _ROWS_PER_SUB)],
                        credp_hbm.at[pl.ds(base + row0, _ROWS_PER_SUB)])
        pltpu.sync_copy(acc_h.at[pl.ds(row0, _ROWS_PER_SUB)],
                        haggp_hbm.at[pl.ds(base + row0, _ROWS_PER_SUB)])
        plsc.subcore_barrier()
        return carry

    lax.fori_loop(0, NP, one_pass, 0)


@jax.jit
def kernel(x, h, c, W_iou, U_iou, b_iou, W_f_w, W_f_b, b_f, U_f_w, U_f_b,
           edge_index):
    R = 1000
    grid = N // R

    bias_f = (W_f_b + b_f[0] + U_f_b).reshape(1, H)
    p_arr, q_arr = pl.pallas_call(
        _prep_body,
        grid=(grid,),
        in_specs=[
            pl.BlockSpec((R, H), lambda i: (i, 0)),
            pl.BlockSpec((R, H), lambda i: (i, 0)),
            pl.BlockSpec((H, H), lambda i: (0, 0)),
            pl.BlockSpec((H, H), lambda i: (0, 0)),
            pl.BlockSpec((1, H), lambda i: (0, 0)),
        ],
        out_specs=[
            pl.BlockSpec((R, H), lambda i: (i, 0)),
            pl.BlockSpec((R, H), lambda i: (i, 0)),
        ],
        out_shape=[
            jax.ShapeDtypeStruct((N, H), jnp.float32),
            jax.ShapeDtypeStruct((N, H), jnp.float32),
        ],
    )(x, h, W_f_w.T, U_f_w.T, bias_f)

    src = edge_index[0].reshape(NS, E // NS)
    dst = edge_index[1].reshape(NS, E // NS)
    pad = ((0, 0), (0, EC - E // NS))
    src_p = jnp.pad(src, pad, constant_values=0)
    dst_p = jnp.pad(dst, pad, constant_values=DUMMY_DST)

    mesh = plsc.VectorSubcoreMesh(core_axis_name="c", subcore_axis_name="s")
    credp, haggp = pl.kernel(
        _edge_kernel,
        out_type=[
            jax.ShapeDtypeStruct((NPAD, H), jnp.float32),
            jax.ShapeDtypeStruct((NPAD, H), jnp.float32),
        ],
        mesh=mesh,
        compiler_params=pltpu.CompilerParams(needs_layout_passes=False),
        scratch_types=[
            pltpu.VMEM((EC,), jnp.int32),
            pltpu.VMEM((EC,), jnp.int32),
            pltpu.VMEM((CMPSZ,), jnp.int32),
            pltpu.VMEM((K,), jnp.int32),
            pltpu.VMEM((K,), jnp.int32),
            pltpu.VMEM((K,), jnp.int32),
            pltpu.VMEM((K, H), jnp.float32),
            pltpu.VMEM((K, H), jnp.float32),
            pltpu.VMEM((K, H), jnp.float32),
            pltpu.VMEM((K, H), jnp.float32),
            pltpu.VMEM((_ZCH, H), jnp.float32),
            pltpu.VMEM((L,), jnp.int32),
            pltpu.VMEM_SHARED((SEG + 8, H), jnp.float32),
            pltpu.VMEM_SHARED((SEG + 8, H), jnp.float32),
            pltpu.SemaphoreType.DMA,
        ],
    )(dst_p, src_p, h, c, p_arr, q_arr)

    c_red = credp[:N]
    h_agg = haggp[:N]

    h_new, c_new = pl.pallas_call(
        _final_body,
        grid=(grid,),
        in_specs=[
            pl.BlockSpec((R, H), lambda i: (i, 0)),
            pl.BlockSpec((R, H), lambda i: (i, 0)),
            pl.BlockSpec((R, H), lambda i: (i, 0)),
            pl.BlockSpec((H, 3 * H), lambda i: (0, 0)),
            pl.BlockSpec((H, 3 * H), lambda i: (0, 0)),
            pl.BlockSpec((1, 3 * H), lambda i: (0, 0)),
        ],
        out_specs=[
            pl.BlockSpec((R, H), lambda i: (i, 0)),
            pl.BlockSpec((R, H), lambda i: (i, 0)),
        ],
        out_shape=[
            jax.ShapeDtypeStruct((N, H), jnp.float32),
            jax.ShapeDtypeStruct((N, H), jnp.float32),
        ],
    )(x, c_red, h_agg, W_iou.T, U_iou.T, b_iou)

    return h_new, c_new


# speedup vs baseline: 1.6363x; 1.0358x over previous
import functools
import jax
import jax.numpy as jnp
from jax import lax
from jax.experimental import pallas as pl
from jax.experimental.pallas import tpu as pltpu
from jax.experimental.pallas import tpu_sc as plsc

N = 100000
H = 128
E = 100000

NC = 2
NS = 16
L = 16
SEG = 5120
NP = 10
NPAD = NP * NC * SEG
EC = 6272
K = 32
DUMMY_DST = 1 << 20
TRASH = EC + K
CMPSZ = EC + K + 16

_ROWS_PER_SUB = SEG // NS
_ZCH = 32


def _prep_body(x_ref, h_ref, wf_ref, uf_ref, bias_ref, p_ref, q_ref):
    p_ref[...] = (
        jnp.dot(x_ref[...], wf_ref[...], precision=lax.Precision.HIGHEST,
                preferred_element_type=jnp.float32)
        + bias_ref[...]
    )
    q_ref[...] = jnp.dot(h_ref[...], uf_ref[...],
                         precision=lax.Precision.HIGHEST,
                         preferred_element_type=jnp.float32)


def _final_body(x_ref, cred_ref, hagg_ref, wiou_ref, uiou_ref, biou_ref,
                hout_ref, cout_ref):
    iou = (
        jnp.dot(x_ref[...], wiou_ref[...], precision=lax.Precision.HIGHEST,
                preferred_element_type=jnp.float32)
        + jnp.dot(hagg_ref[...], uiou_ref[...],
                  precision=lax.Precision.HIGHEST,
                  preferred_element_type=jnp.float32)
        + biou_ref[...]
    )
    i = jax.nn.sigmoid(iou[:, :H])
    o = jax.nn.sigmoid(iou[:, H:2 * H])
    u = jnp.tanh(iou[:, 2 * H:])
    c_new = i * u + cred_ref[...]
    hout_ref[...] = o * jnp.tanh(c_new)
    cout_ref[...] = c_new


def _edge_kernel(dst_hbm, src_hbm, h_hbm, c_hbm, p_hbm, q_hbm,
                 credp_hbm, haggp_hbm,
                 dst_v, src_v, cmp_v,
                 idx_s, idx_d, idx_dl, bh, bc, bq, bp,
                 zeros_v, scr16, acc_c, acc_h, sem):
    cid = lax.axis_index("c")
    sid = lax.axis_index("s")

    pltpu.sync_copy(dst_hbm.at[sid], dst_v)
    pltpu.sync_copy(src_hbm.at[sid], src_v)

    def zinit(i, carry):
        r = i // (H // L)
        col = pl.multiple_of((i % (H // L)) * L, L)
        zeros_v[r, pl.ds(col, L)] = jnp.zeros((L,), jnp.float32)
        return carry

    lax.fori_loop(0, _ZCH * (H // L), zinit, 0)

    def one_pass(p, carry):
        base = (p * NC + cid) * SEG

        for z in range(_ROWS_PER_SUB // _ZCH):
            row0 = sid * _ROWS_PER_SUB + z * _ZCH
            pltpu.sync_copy(zeros_v, acc_c.at[pl.ds(row0, _ZCH)])
            pltpu.sync_copy(zeros_v, acc_h.at[pl.ds(row0, _ZCH)])
        plsc.subcore_barrier()

        def filt(j, cnt):
            off = pl.multiple_of(j * L, L)
            iot = lax.iota(jnp.int32, L)
            ids = iot + off
            d = dst_v[pl.ds(off, L)]
            m = jnp.logical_and(d >= base, d < base + SEG)
            key = jnp.where(m, iot, iot + L)
            _, sv = plsc.sort_key_val(key, ids)
            cmp_v[pl.ds(cnt, L)] = sv
            pc = plsc.all_reduce_population_count(m)
            return cnt + pc[0]

        m_cnt = lax.fori_loop(0, EC // L, filt, jnp.int32(0))

        for t in range(K // L):
            pos = m_cnt + t * L + lax.iota(jnp.int32, L)
            plsc.store_scatter(cmp_v, [pos], jnp.full((L,), EC - 1, jnp.int32))

        n_chunks = (m_cnt + (K - 1)) // K

        def chunk(ck, carry2):
            off = pl.multiple_of(ck * K, K)
            for t in range(K // L):
                ids = cmp_v[pl.ds(off + t * L, L)]
                sidx = plsc.load_gather(src_v, [ids])
                d = plsc.load_gather(dst_v, [ids])
                idx_s[pl.ds(t * L, L)] = sidx
                idx_d[pl.ds(t * L, L)] = jnp.minimum(d, N - 1)
                idx_dl[pl.ds(t * L, L)] = jnp.clip(d - base, 0, SEG)
            cp1 = pltpu.async_copy(h_hbm.at[idx_s], bh, sem)
            cp2 = pltpu.async_copy(c_hbm.at[idx_s], bc, sem)
            cp3 = pltpu.async_copy(q_hbm.at[idx_s], bq, sem)
            cp4 = pltpu.async_copy(p_hbm.at[idx_d], bp, sem)
            cp1.wait()
            cp2.wait()
            cp3.wait()
            cp4.wait()

            def fuse(r, carry3):
                for t in range(H // L):
                    col = t * L
                    pv = bp[r, pl.ds(col, L)]
                    qv = bq[r, pl.ds(col, L)]
                    cv = bc[r, pl.ds(col, L)]
                    f = 1.0 / (1.0 + jnp.exp(-(pv + qv)))
                    bc[r, pl.ds(col, L)] = f * cv
                return carry3

            lax.fori_loop(0, K, fuse, 0)
            pltpu.sync_copy(bc, acc_c.at[idx_dl], add=True)
            pltpu.sync_copy(bh, acc_h.at[idx_dl], add=True)
            return carry2

        lax.fori_loop(0, n_chunks, chunk, 0)
        plsc.subcore_barrier()

        row0 = sid * _ROWS_PER_SUB
        pltpu.sync_copy(acc_c.at[pl.ds(row0, _ROWS_PER_SUB)],
                        credp_hbm.at[pl.ds(base + row0, _ROWS_PER_SUB)])
        pltpu.sync_copy(acc_h.at[pl.ds(row0, _ROWS_PER_SUB)],
                        haggp_hbm.at[pl.ds(base + row0, _ROWS_PER_SUB)])
        plsc.subcore_barrier()
        return carry

    lax.fori_loop(0, NP, one_pass, 0)


@jax.jit
def kernel(x, h, c, W_iou, U_iou, b_iou, W_f_w, W_f_b, b_f, U_f_w, U_f_b,
           edge_index):
    R = 1000
    grid = N // R

    bias_f = (W_f_b + b_f[0] + U_f_b).reshape(1, H)
    p_arr, q_arr = pl.pallas_call(
        _prep_body,
        grid=(grid,),
        in_specs=[
            pl.BlockSpec((R, H), lambda i: (i, 0)),
            pl.BlockSpec((R, H), lambda i: (i, 0)),
            pl.BlockSpec((H, H), lambda i: (0, 0)),
            pl.BlockSpec((H, H), lambda i: (0, 0)),
            pl.BlockSpec((1, H), lambda i: (0, 0)),
        ],
        out_specs=[
            pl.BlockSpec((R, H), lambda i: (i, 0)),
            pl.BlockSpec((R, H), lambda i: (i, 0)),
        ],
        out_shape=[
            jax.ShapeDtypeStruct((N, H), jnp.float32),
            jax.ShapeDtypeStruct((N, H), jnp.float32),
        ],
    )(x, h, W_f_w.T, U_f_w.T, bias_f)

    src = edge_index[0].reshape(NS, E // NS)
    dst = edge_index[1].reshape(NS, E // NS)
    pad = ((0, 0), (0, EC - E // NS))
    src_p = jnp.pad(src, pad, constant_values=0)
    dst_p = jnp.pad(dst, pad, constant_values=DUMMY_DST)

    mesh = plsc.VectorSubcoreMesh(core_axis_name="c", subcore_axis_name="s")
    credp, haggp = pl.kernel(
        _edge_kernel,
        out_type=[
            jax.ShapeDtypeStruct((NPAD, H), jnp.float32),
            jax.ShapeDtypeStruct((NPAD, H), jnp.float32),
        ],
        mesh=mesh,
        compiler_params=pltpu.CompilerParams(needs_layout_passes=False),
        scratch_types=[
            pltpu.VMEM((EC,), jnp.int32),
            pltpu.VMEM((EC,), jnp.int32),
            pltpu.VMEM((CMPSZ,), jnp.int32),
            pltpu.VMEM((K,), jnp.int32),
            pltpu.VMEM((K,), jnp.int32),
            pltpu.VMEM((K,), jnp.int32),
            pltpu.VMEM((K, H), jnp.float32),
            pltpu.VMEM((K, H), jnp.float32),
            pltpu.VMEM((K, H), jnp.float32),
            pltpu.VMEM((K, H), jnp.float32),
            pltpu.VMEM((_ZCH, H), jnp.float32),
            pltpu.VMEM((L,), jnp.int32),
            pltpu.VMEM_SHARED((SEG + 8, H), jnp.float32),
            pltpu.VMEM_SHARED((SEG + 8, H), jnp.float32),
            pltpu.SemaphoreType.DMA,
        ],
    )(dst_p, src_p, h, c, p_arr, q_arr)

    c_red = credp[:N]
    h_agg = haggp[:N]

    h_new, c_new = pl.pallas_call(
        _final_body,
        grid=(grid,),
        in_specs=[
            pl.BlockSpec((R, H), lambda i: (i, 0)),
            pl.BlockSpec((R, H), lambda i: (i, 0)),
            pl.BlockSpec((R, H), lambda i: (i, 0)),
            pl.BlockSpec((H, 3 * H), lambda i: (0, 0)),
            pl.BlockSpec((H, 3 * H), lambda i: (0, 0)),
            pl.BlockSpec((1, 3 * H), lambda i: (0, 0)),
        ],
        out_specs=[
            pl.BlockSpec((R, H), lambda i: (i, 0)),
            pl.BlockSpec((R, H), lambda i: (i, 0)),
        ],
        out_shape=[
            jax.ShapeDtypeStruct((N, H), jnp.float32),
            jax.ShapeDtypeStruct((N, H), jnp.float32),
        ],
    )(x, c_red, h_agg, W_iou.T, U_iou.T, b_iou)

    return h_new, c_new
